# baseline (speedup 1.0000x reference)
import functools

import jax
import jax.numpy as jnp
from jax import lax
from jax.experimental import pallas as pl
from jax.experimental.pallas import tpu as pltpu
from jax.experimental.pallas import tpu_sc as plsc

_NW = 16
_L = 16


def _fifth_root(u):
    um = jnp.maximum(u, 1e-30)
    bits = plsc.bitcast(um, jnp.int32)
    y = plsc.bitcast(bits // 5 + 852282573, jnp.float32)
    for _ in range(4):
        y2 = y * y
        y = 0.2 * (4.0 * y + um / (y2 * y2))
    return jnp.where(u > 0.0, y, 0.0)


def _make_sc_kernel(n_pad, n_pos, n_gt, n_cls):
    chunk = n_pad // _NW
    ppw = n_pos // _NW
    mesh = plsc.VectorSubcoreMesh(
        core_axis_name="c", subcore_axis_name="s", num_cores=1)

    @functools.partial(
        pl.kernel,
        out_type=jax.ShapeDtypeStruct((n_pad,), jnp.float32),
        mesh=mesh,
        compiler_params=pltpu.CompilerParams(
            needs_layout_passes=False, use_tc_tiling_on_sc=False),
        scratch_types=[
            pltpu.VMEM((chunk,), jnp.float32),
            pltpu.VMEM((ppw,), jnp.int32),
            pltpu.VMEM((4 * ppw,), jnp.int32),
            pltpu.VMEM((4 * ppw,), jnp.float32),
            pltpu.VMEM((ppw * n_cls,), jnp.float32),
            pltpu.VMEM((4 * n_gt,), jnp.float32),
            pltpu.VMEM((n_gt,), jnp.int32),
            pltpu.VMEM((ppw,), jnp.float32),
            pltpu.SemaphoreType.DMA,
            pltpu.SemaphoreType.DMA,
            pltpu.SemaphoreType.DMA,
        ],
    )
    def sc_kernel(clsn_flat, bboxn_flat, gtbn_flat, positive_inds, gt_labels,
                  out,
                  zero_v, idx_v, bidx_v, boxc_v, clsf_v, gtb_v, gtl_v,
                  val_v, sem0, sem1, sem2):
        w = lax.axis_index("s")

        def zbody(i, carry):
            zero_v[pl.ds(i * _L, _L)] = jnp.zeros((_L,), jnp.float32)
            return carry
        lax.fori_loop(0, chunk // _L, zbody, 0)
        pltpu.sync_copy(zero_v, out.at[pl.ds(w * chunk, chunk)])
        plsc.subcore_barrier()

        cp_gtb = pltpu.async_copy(gtbn_flat, gtb_v, sem1)
        cp_gtl = pltpu.async_copy(gt_labels, gtl_v, sem2)
        pltpu.sync_copy(positive_inds.at[pl.ds(w * ppw, ppw)], idx_v)
        for c in range(4):
            for k in range(ppw // _L):
                pi = idx_v[pl.ds(k * _L, _L)]
                bidx_v[pl.ds(c * ppw + k * _L, _L)] = pi * 4 + c
        cp_box = pltpu.async_copy(bboxn_flat.at[bidx_v], boxc_v, sem0)
        cls_copies = []
        for k in range(ppw // _L):
            idx_vec = idx_v[pl.ds(k * _L, _L)]
            for j in range(_L):
                p = k * _L + j
                row = idx_vec[j] * n_cls
                cls_copies.append(pltpu.async_copy(
                    clsn_flat.at[pl.ds(row, n_cls)],
                    clsf_v.at[pl.ds(p * n_cls, n_cls)], sem1))
        cp_gtb.wait()
        cp_gtl.wait()
        cp_box.wait()
        for cp in cls_copies:
            cp.wait()
        for k in range(4 * ppw // _L):
            boxc_v[pl.ds(k * _L, _L)] = -boxc_v[pl.ds(k * _L, _L)]
        for k in range(4 * n_gt // _L):
            gtb_v[pl.ds(k * _L, _L)] = -gtb_v[pl.ds(k * _L, _L)]

        lanes = lax.iota(jnp.int32, _L)
        zeros_i = jnp.zeros((_L,), jnp.int32)
        for pb in range(ppw // _L):
            ax1 = boxc_v[pl.ds(0 * ppw + pb * _L, _L)]
            ay1 = boxc_v[pl.ds(1 * ppw + pb * _L, _L)]
            ax2 = boxc_v[pl.ds(2 * ppw + pb * _L, _L)]
            ay2 = boxc_v[pl.ds(3 * ppw + pb * _L, _L)]
            area1 = (ax2 - ax1) * (ay2 - ay1)
            cls_base = (lanes + pb * _L) * n_cls

            def one_gt(g):
                lab = plsc.load_gather(gtl_v, [zeros_i + g])
                gx1 = plsc.load_gather(gtb_v, [zeros_i + g * 4])
                gy1 = plsc.load_gather(gtb_v, [zeros_i + (g * 4 + 1)])
                gx2 = plsc.load_gather(gtb_v, [zeros_i + (g * 4 + 2)])
                gy2 = plsc.load_gather(gtb_v, [zeros_i + (g * 4 + 3)])
                sneg = plsc.load_gather(clsf_v, [cls_base + lab])
                s = 1.0 / (1.0 + jnp.exp(sneg))
                area2 = (gx2 - gx1) * (gy2 - gy1)
                iw = jnp.maximum(
                    jnp.minimum(ax2, gx2) - jnp.maximum(ax1, gx1), 0.0)
                ih = jnp.maximum(
                    jnp.minimum(ay2, gy2) - jnp.maximum(ay1, gy1), 0.0)
                inter = iw * ih
                union = jnp.maximum(area1 + area2 - inter, 1e-6)
                iou = inter / union
                iou2 = iou * iou
                return s * (iou2 * iou2)

            def gbody(i, m):
                us = [one_gt(i * 4 + j) for j in range(4)]
                u01 = jnp.maximum(us[0], us[1])
                u23 = jnp.maximum(us[2], us[3])
                return jnp.maximum(m, jnp.maximum(u01, u23))

            assert n_gt % 4 == 0
            m = lax.fori_loop(0, n_gt // 4, gbody,
                              jnp.zeros((_L,), jnp.float32))
            val_v[pl.ds(pb * _L, _L)] = _fifth_root(m)

        pltpu.async_copy(val_v, out.at[idx_v], sem0).wait()

    return sc_kernel


def kernel(bboxes, cls_scores, bbox_preds, gt_bboxes, bbox_levels,
           positive_inds, gt_labels):
    del bboxes, bbox_levels
    n, n_cls = cls_scores.shape
    n_pos = positive_inds.shape[0]
    n_gt = gt_bboxes.shape[0]
    chunk = -(-n // (_NW * _L)) * _L
    sc_kernel = _make_sc_kernel(chunk * _NW, n_pos, n_gt, n_cls)
    out = sc_kernel((-cls_scores).reshape(-1),
                    (-bbox_preds[:, :4]).reshape(-1),
                    (-gt_bboxes[:, :4]).reshape(-1),
                    positive_inds.astype(jnp.int32),
                    gt_labels.astype(jnp.int32))
    return out[:n]


# speedup vs baseline: 1.4066x; 1.4066x over previous
import functools

import jax
import jax.numpy as jnp
from jax import lax
from jax.experimental import pallas as pl
from jax.experimental.pallas import tpu as pltpu
from jax.experimental.pallas import tpu_sc as plsc

_NW = 16
_L = 16
_ROW = 128


def _fifth_root(u):
    um = jnp.maximum(u, 1e-30)
    bits = plsc.bitcast(um, jnp.int32)
    y = plsc.bitcast(bits // 5 + 852282573, jnp.float32)
    for _ in range(4):
        y2 = y * y
        y = 0.2 * (4.0 * y + um / (y2 * y2))
    return jnp.where(u > 0.0, y, 0.0)


def _stage_rows(cls_scores, bbox4):
    n, n_cls = cls_scores.shape
    rows = 2000
    assert n % rows == 0

    def body(c_ref, b_ref, o_ref):
        o_ref[:, 0:n_cls] = c_ref[...]
        o_ref[:, n_cls:n_cls + 4] = b_ref[...]

    return pl.pallas_call(
        body,
        grid=(n // rows,),
        in_specs=[
            pl.BlockSpec((rows, n_cls), lambda i: (i, 0)),
            pl.BlockSpec((rows, 4), lambda i: (i, 0)),
        ],
        out_specs=pl.BlockSpec((rows, _ROW), lambda i: (i, 0)),
        out_shape=jax.ShapeDtypeStruct((n, _ROW), jnp.float32),
    )(cls_scores, bbox4)


def _make_sc_kernel(n_pad, n_pos, n_gt, n_cls):
    chunk = n_pad // _NW
    ppw = n_pos // _NW
    mesh = plsc.VectorSubcoreMesh(
        core_axis_name="c", subcore_axis_name="s", num_cores=1)

    @functools.partial(
        pl.kernel,
        out_type=jax.ShapeDtypeStruct((n_pad,), jnp.float32),
        mesh=mesh,
        compiler_params=pltpu.CompilerParams(
            needs_layout_passes=False, use_tc_tiling_on_sc=False),
        scratch_types=[
            pltpu.VMEM((chunk,), jnp.float32),
            pltpu.VMEM((ppw,), jnp.int32),
            pltpu.VMEM((4 * ppw,), jnp.int32),
            pltpu.VMEM((4 * ppw,), jnp.float32),
            pltpu.VMEM((ppw * n_cls,), jnp.float32),
            pltpu.VMEM((4 * n_gt,), jnp.float32),
            pltpu.VMEM((n_gt,), jnp.int32),
            pltpu.VMEM((ppw,), jnp.float32),
            pltpu.SemaphoreType.DMA,
            pltpu.SemaphoreType.DMA,
            pltpu.SemaphoreType.DMA,
        ],
    )
    def sc_kernel(staged_flat, gtbn_flat, positive_inds, gt_labels,
                  out,
                  zero_v, idx_v, bidx_v, boxc_v, clsf_v, gtb_v, gtl_v,
                  val_v, sem0, sem1, sem2):
        w = lax.axis_index("s")

        def zbody(i, carry):
            zero_v[pl.ds(i * _L, _L)] = jnp.zeros((_L,), jnp.float32)
            return carry
        lax.fori_loop(0, chunk // _L, zbody, 0)
        pltpu.sync_copy(zero_v, out.at[pl.ds(w * chunk, chunk)])
        plsc.subcore_barrier()

        cp_gtb = pltpu.async_copy(gtbn_flat, gtb_v, sem1)
        cp_gtl = pltpu.async_copy(gt_labels, gtl_v, sem2)
        pltpu.sync_copy(positive_inds.at[pl.ds(w * ppw, ppw)], idx_v)
        for c in range(4):
            for k in range(ppw // _L):
                pi = idx_v[pl.ds(k * _L, _L)]
                bidx_v[pl.ds(c * ppw + k * _L, _L)] = \
                    pi * _ROW + (n_cls + c)
        cp_box = pltpu.async_copy(staged_flat.at[bidx_v], boxc_v, sem0)
        cls_copies = []
        for k in range(ppw // _L):
            idx_vec = idx_v[pl.ds(k * _L, _L)]
            for j in range(_L):
                p = k * _L + j
                row = idx_vec[j] * _ROW
                cls_copies.append(pltpu.async_copy(
                    staged_flat.at[pl.ds(row, n_cls)],
                    clsf_v.at[pl.ds(p * n_cls, n_cls)], sem1))
        cp_gtb.wait()
        cp_gtl.wait()
        cp_box.wait()
        for cp in cls_copies:
            cp.wait()
        for k in range(4 * n_gt // _L):
            gtb_v[pl.ds(k * _L, _L)] = -gtb_v[pl.ds(k * _L, _L)]

        lanes = lax.iota(jnp.int32, _L)
        zeros_i = jnp.zeros((_L,), jnp.int32)
        for pb in range(ppw // _L):
            ax1 = boxc_v[pl.ds(0 * ppw + pb * _L, _L)]
            ay1 = boxc_v[pl.ds(1 * ppw + pb * _L, _L)]
            ax2 = boxc_v[pl.ds(2 * ppw + pb * _L, _L)]
            ay2 = boxc_v[pl.ds(3 * ppw + pb * _L, _L)]
            area1 = (ax2 - ax1) * (ay2 - ay1)
            cls_base = (lanes + pb * _L) * n_cls

            def one_gt(g):
                lab = plsc.load_gather(gtl_v, [zeros_i + g])
                gx1 = plsc.load_gather(gtb_v, [zeros_i + g * 4])
                gy1 = plsc.load_gather(gtb_v, [zeros_i + (g * 4 + 1)])
                gx2 = plsc.load_gather(gtb_v, [zeros_i + (g * 4 + 2)])
                gy2 = plsc.load_gather(gtb_v, [zeros_i + (g * 4 + 3)])
                sraw = plsc.load_gather(clsf_v, [cls_base + lab])
                s = 1.0 / (1.0 + jnp.exp(-sraw))
                area2 = (gx2 - gx1) * (gy2 - gy1)
                iw = jnp.maximum(
                    jnp.minimum(ax2, gx2) - jnp.maximum(ax1, gx1), 0.0)
                ih = jnp.maximum(
                    jnp.minimum(ay2, gy2) - jnp.maximum(ay1, gy1), 0.0)
                inter = iw * ih
                union = jnp.maximum(area1 + area2 - inter, 1e-6)
                iou = inter / union
                iou2 = iou * iou
                return s * (iou2 * iou2)

            def gbody(i, m):
                us = [one_gt(i * 4 + j) for j in range(4)]
                u01 = jnp.maximum(us[0], us[1])
                u23 = jnp.maximum(us[2], us[3])
                return jnp.maximum(m, jnp.maximum(u01, u23))

            assert n_gt % 4 == 0
            m = lax.fori_loop(0, n_gt // 4, gbody,
                              jnp.zeros((_L,), jnp.float32))
            val_v[pl.ds(pb * _L, _L)] = _fifth_root(m)

        pltpu.async_copy(val_v, out.at[idx_v], sem0).wait()

    return sc_kernel


def kernel(bboxes, cls_scores, bbox_preds, gt_bboxes, bbox_levels,
           positive_inds, gt_labels):
    del bboxes, bbox_levels
    n, n_cls = cls_scores.shape
    n_pos = positive_inds.shape[0]
    n_gt = gt_bboxes.shape[0]
    chunk = -(-n // (_NW * _L)) * _L
    staged = _stage_rows(cls_scores, bbox_preds[:, :4])
    sc_kernel = _make_sc_kernel(chunk * _NW, n_pos, n_gt, n_cls)
    out = sc_kernel(staged.reshape(-1),
                    (-gt_bboxes[:, :4]).reshape(-1),
                    positive_inds.astype(jnp.int32),
                    gt_labels.astype(jnp.int32))
    return out[:n]


# speedup vs baseline: 1.8899x; 1.3436x over previous
import functools

import jax
import jax.numpy as jnp
from jax import lax
from jax.experimental import pallas as pl
from jax.experimental.pallas import tpu as pltpu
from jax.experimental.pallas import tpu_sc as plsc

_NW = 16
_L = 16


def _fifth_root(u):
    um = jnp.maximum(u, 1e-30)
    bits = plsc.bitcast(um, jnp.int32)
    y = plsc.bitcast(bits // 5 + 852282573, jnp.float32)
    for _ in range(4):
        y2 = y * y
        y = 0.2 * (4.0 * y + um / (y2 * y2))
    return jnp.where(u > 0.0, y, 0.0)


def _stage(cls_t, bbox_t, gtb_t, stride):
    n_cls, n = cls_t.shape
    n_gt = gtb_t.shape[1]

    def body(c_ref, b_ref, g_ref, o_ref):
        o_ref[0:n_cls, 0:n] = c_ref[...]
        o_ref[n_cls:n_cls + 4, 0:n] = b_ref[...]
        o_ref[88:92, 0:n_gt] = g_ref[...]

    return pl.pallas_call(
        body,
        out_shape=jax.ShapeDtypeStruct((96, stride), jnp.float32),
    )(cls_t, bbox_t, gtb_t)


def _make_sc_kernel(n_pad, stride, n_pos, n_gt, n_cls):
    chunk = n_pad // _NW
    ppw = n_pos // _NW
    mesh = plsc.VectorSubcoreMesh(
        core_axis_name="c", subcore_axis_name="s", num_cores=1)

    @functools.partial(
        pl.kernel,
        out_type=jax.ShapeDtypeStruct((n_pad,), jnp.float32),
        mesh=mesh,
        compiler_params=pltpu.CompilerParams(
            needs_layout_passes=False, use_tc_tiling_on_sc=False),
        scratch_types=[
            pltpu.VMEM((chunk,), jnp.float32),
            pltpu.VMEM((ppw,), jnp.int32),
            pltpu.VMEM((4 * ppw,), jnp.int32),
            pltpu.VMEM((4 * ppw,), jnp.float32),
            pltpu.VMEM((ppw * n_cls,), jnp.int32),
            pltpu.VMEM((ppw * n_cls,), jnp.float32),
            pltpu.VMEM((4 * 104,), jnp.float32),
            pltpu.VMEM((n_gt,), jnp.int32),
            pltpu.VMEM((ppw,), jnp.float32),
            pltpu.SemaphoreType.DMA,
            pltpu.SemaphoreType.DMA,
            pltpu.SemaphoreType.DMA,
        ],
    )
    def sc_kernel(staged_flat, positive_inds, gt_labels,
                  out,
                  zero_v, idx_v, bidx_v, boxc_v, cidx_v, clsf_v, gtb_v,
                  gtl_v, val_v, sem0, sem1, sem2):
        w = lax.axis_index("s")

        def zbody(i, carry):
            zero_v[pl.ds(i * _L, _L)] = jnp.zeros((_L,), jnp.float32)
            return carry
        lax.fori_loop(0, chunk // _L, zbody, 0)
        pltpu.sync_copy(zero_v, out.at[pl.ds(w * chunk, chunk)])
        plsc.subcore_barrier()

        cp_gtl = pltpu.async_copy(gt_labels, gtl_v, sem2)
        gt_copies = [
            pltpu.async_copy(staged_flat.at[pl.ds((88 + c) * stride, n_gt)],
                             gtb_v.at[pl.ds(c * 104, n_gt)], sem2)
            for c in range(4)]
        pltpu.sync_copy(positive_inds.at[pl.ds(w * ppw, ppw)], idx_v)
        idx_lo = idx_v[pl.ds(0, _L)]
        idx_hi = idx_v[pl.ds(_L, _L)]
        for c in range(4):
            base = (n_cls + c) * stride
            bidx_v[pl.ds(c * ppw, _L)] = idx_lo + base
            bidx_v[pl.ds(c * ppw + _L, _L)] = idx_hi + base
        cp_box = pltpu.async_copy(staged_flat.at[bidx_v], boxc_v, sem0)
        def cbody(c, carry):
            cidx_v[pl.ds(c * ppw, _L)] = idx_lo + c * stride
            cidx_v[pl.ds(c * ppw + _L, _L)] = idx_hi + c * stride
            return carry
        lax.fori_loop(0, n_cls, cbody, 0)
        cls_copies = []
        n_chunk = ppw * n_cls // 128
        for k in range(n_chunk):
            cls_copies.append(pltpu.async_copy(
                staged_flat.at[cidx_v.at[pl.ds(k * 128, 128)]],
                clsf_v.at[pl.ds(k * 128, 128)], sem1))
        cp_gtl.wait()
        for cp in gt_copies:
            cp.wait()
        cp_box.wait()
        for cp in cls_copies:
            cp.wait()

        lanes = lax.iota(jnp.int32, _L)
        zeros_i = jnp.zeros((_L,), jnp.int32)
        for pb in range(ppw // _L):
            ax1 = boxc_v[pl.ds(0 * ppw + pb * _L, _L)]
            ay1 = boxc_v[pl.ds(1 * ppw + pb * _L, _L)]
            ax2 = boxc_v[pl.ds(2 * ppw + pb * _L, _L)]
            ay2 = boxc_v[pl.ds(3 * ppw + pb * _L, _L)]
            area1 = (ax2 - ax1) * (ay2 - ay1)
            pidx = lanes + pb * _L

            def one_gt(g):
                lab = plsc.load_gather(gtl_v, [zeros_i + g])
                gx1 = plsc.load_gather(gtb_v, [zeros_i + g])
                gy1 = plsc.load_gather(gtb_v, [zeros_i + (104 + g)])
                gx2 = plsc.load_gather(gtb_v, [zeros_i + (2 * 104 + g)])
                gy2 = plsc.load_gather(gtb_v, [zeros_i + (3 * 104 + g)])
                sraw = plsc.load_gather(clsf_v, [lab * ppw + pidx])
                s = 1.0 / (1.0 + jnp.exp(-sraw))
                area2 = (gx2 - gx1) * (gy2 - gy1)
                iw = jnp.maximum(
                    jnp.minimum(ax2, gx2) - jnp.maximum(ax1, gx1), 0.0)
                ih = jnp.maximum(
                    jnp.minimum(ay2, gy2) - jnp.maximum(ay1, gy1), 0.0)
                inter = iw * ih
                union = jnp.maximum(area1 + area2 - inter, 1e-6)
                iou = inter / union
                iou2 = iou * iou
                return s * (iou2 * iou2)

            def gbody(i, m):
                us = [one_gt(i * 4 + j) for j in range(4)]
                u01 = jnp.maximum(us[0], us[1])
                u23 = jnp.maximum(us[2], us[3])
                return jnp.maximum(m, jnp.maximum(u01, u23))

            assert n_gt % 4 == 0
            m = lax.fori_loop(0, n_gt // 4, gbody,
                              jnp.zeros((_L,), jnp.float32))
            val_v[pl.ds(pb * _L, _L)] = _fifth_root(m)

        pltpu.async_copy(val_v, out.at[idx_v], sem0).wait()

    return sc_kernel


def kernel(bboxes, cls_scores, bbox_preds, gt_bboxes, bbox_levels,
           positive_inds, gt_labels):
    del bboxes, bbox_levels
    n, n_cls = cls_scores.shape
    n_pos = positive_inds.shape[0]
    n_gt = gt_bboxes.shape[0]
    stride = -(-n // 128) * 128
    chunk = -(-n // (_NW * _L)) * _L
    staged = _stage(cls_scores.T, bbox_preds[:, :4].T, gt_bboxes[:, :4].T,
                    stride)
    sc_kernel = _make_sc_kernel(chunk * _NW, stride, n_pos, n_gt, n_cls)
    out = sc_kernel(staged.reshape(-1),
                    positive_inds.astype(jnp.int32),
                    gt_labels.astype(jnp.int32))
    return out[:n]


# speedup vs baseline: 1.9849x; 1.0503x over previous
import functools

import jax
import jax.numpy as jnp
from jax import lax
from jax.experimental import pallas as pl
from jax.experimental.pallas import tpu as pltpu
from jax.experimental.pallas import tpu_sc as plsc

_NW = 16
_L = 16
_ROW = 128


def _fifth_root(u):
    um = jnp.maximum(u, 1e-30)
    bits = plsc.bitcast(um, jnp.int32)
    y = plsc.bitcast(bits // 5 + 852282573, jnp.float32)
    for _ in range(4):
        y2 = y * y
        y = 0.2 * (4.0 * y + um / (y2 * y2))
    return jnp.where(u > 0.0, y, 0.0)


def _stage(cls_t, bbox_t, gtb_t):
    n_cls, n = cls_t.shape
    n_gt = gtb_t.shape[1]
    gt_rows = -(-n_gt // 8) * 8
    cols = 2048
    grid = -(-n // cols)

    def body(c_ref, b_ref, g_ref, o_ref, og_ref):
        o_ref[:, 0:n_cls] = jnp.transpose(c_ref[...])
        o_ref[:, n_cls:n_cls + 4] = jnp.transpose(b_ref[...])

        @pl.when(pl.program_id(0) == 0)
        def _():
            og_ref[0:n_gt, 0:4] = jnp.transpose(g_ref[...])

    return pl.pallas_call(
        body,
        grid=(grid,),
        in_specs=[
            pl.BlockSpec((n_cls, cols), lambda i: (0, i)),
            pl.BlockSpec((4, cols), lambda i: (0, i)),
            pl.BlockSpec((4, n_gt), lambda i: (0, 0)),
        ],
        out_specs=[
            pl.BlockSpec((cols, _ROW), lambda i: (i, 0)),
            pl.BlockSpec((gt_rows, _ROW), lambda i: (0, 0)),
        ],
        out_shape=[
            jax.ShapeDtypeStruct((grid * cols, _ROW), jnp.float32),
            jax.ShapeDtypeStruct((gt_rows, _ROW), jnp.float32),
        ],
    )(cls_t, bbox_t, gtb_t)


def _make_sc_kernel(n, n_pos, n_gt, n_cls, n_staged):
    chunk = -(-n // (_NW * _L)) * _L
    tail = n - chunk * (_NW - 1)
    assert 0 < tail <= chunk and tail % 8 == 0
    ppw = n_pos // _NW
    mesh = plsc.VectorSubcoreMesh(
        core_axis_name="c", subcore_axis_name="s", num_cores=1)

    @functools.partial(
        pl.kernel,
        out_type=jax.ShapeDtypeStruct((n,), jnp.float32),
        mesh=mesh,
        compiler_params=pltpu.CompilerParams(
            needs_layout_passes=False, use_tc_tiling_on_sc=False),
        scratch_types=[
            pltpu.VMEM((chunk,), jnp.float32),
            pltpu.VMEM((ppw,), jnp.int32),
            pltpu.VMEM((4 * ppw,), jnp.int32),
            pltpu.VMEM((4 * ppw,), jnp.float32),
            pltpu.VMEM((ppw * n_cls,), jnp.float32),
            pltpu.VMEM((n_gt * _ROW,), jnp.float32),
            pltpu.VMEM((n_gt,), jnp.int32),
            pltpu.VMEM((ppw,), jnp.float32),
            pltpu.SemaphoreType.DMA,
            pltpu.SemaphoreType.DMA,
            pltpu.SemaphoreType.DMA,
        ],
    )
    def sc_kernel(staged_flat, gt_flat, positive_inds, gt_labels,
                  out,
                  zero_v, idx_v, bidx_v, boxc_v, clsf_v, gtb_v, gtl_v,
                  val_v, sem0, sem1, sem2):
        w = lax.axis_index("s")

        def zbody(i, carry):
            zero_v[pl.ds(i * _L, _L)] = jnp.zeros((_L,), jnp.float32)
            return carry
        lax.fori_loop(0, chunk // _L, zbody, 0)

        @pl.when(w < _NW - 1)
        def _():
            pltpu.sync_copy(zero_v, out.at[pl.ds(w * chunk, chunk)])

        @pl.when(w == _NW - 1)
        def _():
            pltpu.sync_copy(zero_v.at[pl.ds(0, tail)],
                            out.at[pl.ds((_NW - 1) * chunk, tail)])

        plsc.subcore_barrier()

        cp_gtl = pltpu.async_copy(gt_labels, gtl_v, sem2)
        cp_gtb = pltpu.async_copy(gt_flat.at[pl.ds(0, n_gt * _ROW)],
                                  gtb_v, sem2)
        pltpu.sync_copy(positive_inds.at[pl.ds(w * ppw, ppw)], idx_v)
        idx_lo = idx_v[pl.ds(0, _L)]
        idx_hi = idx_v[pl.ds(_L, _L)]
        for c in range(4):
            bidx_v[pl.ds(c * ppw, _L)] = idx_lo * _ROW + (n_cls + c)
            bidx_v[pl.ds(c * ppw + _L, _L)] = idx_hi * _ROW + (n_cls + c)
        cp_box = pltpu.async_copy(staged_flat.at[bidx_v], boxc_v, sem0)
        cls_copies = []
        for k in range(ppw // _L):
            idx_vec = idx_v[pl.ds(k * _L, _L)]
            for j in range(_L):
                p = k * _L + j
                cls_copies.append(pltpu.async_copy(
                    staged_flat.at[pl.ds(idx_vec[j] * _ROW, n_cls)],
                    clsf_v.at[pl.ds(p * n_cls, n_cls)], sem1))
        cp_gtl.wait()
        cp_gtb.wait()
        cp_box.wait()
        for cp in cls_copies:
            cp.wait()

        lanes = lax.iota(jnp.int32, _L)
        zeros_i = jnp.zeros((_L,), jnp.int32)
        for pb in range(ppw // _L):
            ax1 = boxc_v[pl.ds(0 * ppw + pb * _L, _L)]
            ay1 = boxc_v[pl.ds(1 * ppw + pb * _L, _L)]
            ax2 = boxc_v[pl.ds(2 * ppw + pb * _L, _L)]
            ay2 = boxc_v[pl.ds(3 * ppw + pb * _L, _L)]
            area1 = (ax2 - ax1) * (ay2 - ay1)
            cls_base = (lanes + pb * _L) * n_cls

            def one_gt(g):
                lab = plsc.load_gather(gtl_v, [zeros_i + g])
                gx1 = plsc.load_gather(gtb_v, [zeros_i + g * _ROW])
                gy1 = plsc.load_gather(gtb_v, [zeros_i + (g * _ROW + 1)])
                gx2 = plsc.load_gather(gtb_v, [zeros_i + (g * _ROW + 2)])
                gy2 = plsc.load_gather(gtb_v, [zeros_i + (g * _ROW + 3)])
                sraw = plsc.load_gather(clsf_v, [cls_base + lab])
                s = 1.0 / (1.0 + jnp.exp(-sraw))
                area2 = (gx2 - gx1) * (gy2 - gy1)
                iw = jnp.maximum(
                    jnp.minimum(ax2, gx2) - jnp.maximum(ax1, gx1), 0.0)
                ih = jnp.maximum(
                    jnp.minimum(ay2, gy2) - jnp.maximum(ay1, gy1), 0.0)
                inter = iw * ih
                union = jnp.maximum(area1 + area2 - inter, 1e-6)
                iou = inter / union
                iou2 = iou * iou
                return s * (iou2 * iou2)

            def gbody(i, m):
                us = [one_gt(i * 4 + j) for j in range(4)]
                u01 = jnp.maximum(us[0], us[1])
                u23 = jnp.maximum(us[2], us[3])
                return jnp.maximum(m, jnp.maximum(u01, u23))

            assert n_gt % 4 == 0
            m = lax.fori_loop(0, n_gt // 4, gbody,
                              jnp.zeros((_L,), jnp.float32))
            val_v[pl.ds(pb * _L, _L)] = _fifth_root(m)

        pltpu.async_copy(val_v, out.at[idx_v], sem0).wait()

    return sc_kernel


def kernel(bboxes, cls_scores, bbox_preds, gt_bboxes, bbox_levels,
           positive_inds, gt_labels):
    del bboxes, bbox_levels
    n, n_cls = cls_scores.shape
    n_pos = positive_inds.shape[0]
    n_gt = gt_bboxes.shape[0]
    staged, gt_staged = _stage(cls_scores.T, bbox_preds[:, :4].T,
                               gt_bboxes[:, :4].T)
    sc_kernel = _make_sc_kernel(n, n_pos, n_gt, n_cls, staged.shape[0])
    return sc_kernel(staged.reshape(-1),
                     gt_staged.reshape(-1),
                     positive_inds.astype(jnp.int32),
                     gt_labels.astype(jnp.int32))


# speedup vs baseline: 2.1166x; 1.0663x over previous
import functools

import jax
import jax.numpy as jnp
from jax import lax
from jax.experimental import pallas as pl
from jax.experimental.pallas import tpu as pltpu
from jax.experimental.pallas import tpu_sc as plsc

_NW = 16
_L = 16


def _fifth_root(u):
    um = jnp.maximum(u, 1e-30)
    bits = plsc.bitcast(um, jnp.int32)
    y = plsc.bitcast(bits // 5 + 852282573, jnp.float32)
    for _ in range(4):
        y2 = y * y
        y = 0.2 * (4.0 * y + um / (y2 * y2))
    return jnp.where(u > 0.0, y, 0.0)


def _stage(cls_t, bbox_t, gtb_t, stride):
    n_cls, n = cls_t.shape
    n_gt = gtb_t.shape[1]
    cls_steps = n_cls // 8
    grid = cls_steps + 2

    def body(c_ref, b_ref, g_ref, o_ref):
        i = pl.program_id(0)

        @pl.when(i < cls_steps)
        def _():
            for k in range(8):
                o_ref[pl.ds(k * stride, n)] = c_ref[k, :]

        @pl.when(i == cls_steps)
        def _():
            for k in range(4):
                o_ref[pl.ds(k * stride, n)] = b_ref[k, :]

        @pl.when(i == cls_steps + 1)
        def _():
            for k in range(4):
                o_ref[pl.ds(k * stride, n_gt)] = g_ref[k, :]

    return pl.pallas_call(
        body,
        grid=(grid,),
        in_specs=[
            pl.BlockSpec((8, n), lambda i: (jnp.minimum(i, cls_steps - 1), 0)),
            pl.BlockSpec((4, n), lambda i: (0, 0)),
            pl.BlockSpec((4, n_gt), lambda i: (0, 0)),
        ],
        out_specs=pl.BlockSpec((8 * stride,), lambda i: (i,)),
        out_shape=jax.ShapeDtypeStruct((grid * 8 * stride,), jnp.float32),
    )(cls_t, bbox_t, gtb_t)


def _make_sc_kernel(n, stride, n_pos, n_gt, n_cls):
    chunk = -(-n // (_NW * _L)) * _L
    tail = n - chunk * (_NW - 1)
    assert 0 < tail <= chunk and tail % 8 == 0
    ppw = n_pos // _NW
    mesh = plsc.VectorSubcoreMesh(
        core_axis_name="c", subcore_axis_name="s", num_cores=1)

    @functools.partial(
        pl.kernel,
        out_type=jax.ShapeDtypeStruct((n,), jnp.float32),
        mesh=mesh,
        compiler_params=pltpu.CompilerParams(
            needs_layout_passes=False, use_tc_tiling_on_sc=False),
        scratch_types=[
            pltpu.VMEM((chunk,), jnp.float32),
            pltpu.VMEM((ppw,), jnp.int32),
            pltpu.VMEM((4 * ppw,), jnp.int32),
            pltpu.VMEM((4 * ppw,), jnp.float32),
            pltpu.VMEM((ppw * n_cls,), jnp.int32),
            pltpu.VMEM((ppw * n_cls,), jnp.float32),
            pltpu.VMEM((4 * 104,), jnp.float32),
            pltpu.VMEM((n_gt,), jnp.int32),
            pltpu.VMEM((ppw,), jnp.float32),
            pltpu.SemaphoreType.DMA,
            pltpu.SemaphoreType.DMA,
            pltpu.SemaphoreType.DMA,
        ],
    )
    def sc_kernel(staged_flat, positive_inds, gt_labels,
                  out,
                  zero_v, idx_v, bidx_v, boxc_v, cidx_v, clsf_v, gtb_v,
                  gtl_v, val_v, sem0, sem1, sem2):
        w = lax.axis_index("s")

        def zbody(i, carry):
            zero_v[pl.ds(i * _L, _L)] = jnp.zeros((_L,), jnp.float32)
            return carry
        lax.fori_loop(0, chunk // _L, zbody, 0)

        @pl.when(w < _NW - 1)
        def _():
            pltpu.sync_copy(zero_v, out.at[pl.ds(w * chunk, chunk)])

        @pl.when(w == _NW - 1)
        def _():
            pltpu.sync_copy(zero_v.at[pl.ds(0, tail)],
                            out.at[pl.ds((_NW - 1) * chunk, tail)])

        plsc.subcore_barrier()

        cp_gtl = pltpu.async_copy(gt_labels, gtl_v, sem2)
        gt_copies = [
            pltpu.async_copy(staged_flat.at[pl.ds((88 + c) * stride, n_gt)],
                             gtb_v.at[pl.ds(c * 104, n_gt)], sem2)
            for c in range(4)]
        pltpu.sync_copy(positive_inds.at[pl.ds(w * ppw, ppw)], idx_v)
        idx_lo = idx_v[pl.ds(0, _L)]
        idx_hi = idx_v[pl.ds(_L, _L)]
        for c in range(4):
            base = (n_cls + c) * stride
            bidx_v[pl.ds(c * ppw, _L)] = idx_lo + base
            bidx_v[pl.ds(c * ppw + _L, _L)] = idx_hi + base
        cp_box = pltpu.async_copy(staged_flat.at[bidx_v], boxc_v, sem0)
        def cbody(c, carry):
            cidx_v[pl.ds(c * ppw, _L)] = idx_lo + c * stride
            cidx_v[pl.ds(c * ppw + _L, _L)] = idx_hi + c * stride
            return carry
        lax.fori_loop(0, n_cls, cbody, 0)
        cls_copies = []
        n_chunk = ppw * n_cls // 128
        for k in range(n_chunk):
            cls_copies.append(pltpu.async_copy(
                staged_flat.at[cidx_v.at[pl.ds(k * 128, 128)]],
                clsf_v.at[pl.ds(k * 128, 128)], sem1))
        cp_gtl.wait()
        for cp in gt_copies:
            cp.wait()
        cp_box.wait()
        for cp in cls_copies:
            cp.wait()

        lanes = lax.iota(jnp.int32, _L)
        zeros_i = jnp.zeros((_L,), jnp.int32)
        for pb in range(ppw // _L):
            ax1 = boxc_v[pl.ds(0 * ppw + pb * _L, _L)]
            ay1 = boxc_v[pl.ds(1 * ppw + pb * _L, _L)]
            ax2 = boxc_v[pl.ds(2 * ppw + pb * _L, _L)]
            ay2 = boxc_v[pl.ds(3 * ppw + pb * _L, _L)]
            area1 = (ax2 - ax1) * (ay2 - ay1)
            pidx = lanes + pb * _L

            def one_gt(g):
                lab = plsc.load_gather(gtl_v, [zeros_i + g])
                gx1 = plsc.load_gather(gtb_v, [zeros_i + g])
                gy1 = plsc.load_gather(gtb_v, [zeros_i + (104 + g)])
                gx2 = plsc.load_gather(gtb_v, [zeros_i + (2 * 104 + g)])
                gy2 = plsc.load_gather(gtb_v, [zeros_i + (3 * 104 + g)])
                sraw = plsc.load_gather(clsf_v, [lab * ppw + pidx])
                s = 1.0 / (1.0 + jnp.exp(-sraw))
                area2 = (gx2 - gx1) * (gy2 - gy1)
                iw = jnp.maximum(
                    jnp.minimum(ax2, gx2) - jnp.maximum(ax1, gx1), 0.0)
                ih = jnp.maximum(
                    jnp.minimum(ay2, gy2) - jnp.maximum(ay1, gy1), 0.0)
                inter = iw * ih
                union = jnp.maximum(area1 + area2 - inter, 1e-6)
                iou = inter / union
                iou2 = iou * iou
                return s * (iou2 * iou2)

            def gbody(i, m):
                us = [one_gt(i * 4 + j) for j in range(4)]
                u01 = jnp.maximum(us[0], us[1])
                u23 = jnp.maximum(us[2], us[3])
                return jnp.maximum(m, jnp.maximum(u01, u23))

            assert n_gt % 4 == 0
            m = lax.fori_loop(0, n_gt // 4, gbody,
                              jnp.zeros((_L,), jnp.float32))
            val_v[pl.ds(pb * _L, _L)] = _fifth_root(m)

        pltpu.async_copy(val_v, out.at[idx_v], sem0).wait()

    return sc_kernel


def kernel(bboxes, cls_scores, bbox_preds, gt_bboxes, bbox_levels,
           positive_inds, gt_labels):
    del bboxes, bbox_levels
    n, n_cls = cls_scores.shape
    n_pos = positive_inds.shape[0]
    n_gt = gt_bboxes.shape[0]
    stride = -(-n // 128) * 128
    staged = _stage(cls_scores.T, bbox_preds[:, :4].T, gt_bboxes[:, :4].T,
                    stride)
    sc_kernel = _make_sc_kernel(n, stride, n_pos, n_gt, n_cls)
    return sc_kernel(staged,
                     positive_inds.astype(jnp.int32),
                     gt_labels.astype(jnp.int32))


# speedup vs baseline: 2.2491x; 1.0626x over previous
import functools

import jax
import jax.numpy as jnp
from jax import lax
from jax.experimental import pallas as pl
from jax.experimental.pallas import tpu as pltpu
from jax.experimental.pallas import tpu_sc as plsc

_NW = 16
_L = 16


def _fifth_root(u):
    um = jnp.maximum(u, 1e-30)
    bits = plsc.bitcast(um, jnp.int32)
    y = plsc.bitcast(bits // 5 + 852282573, jnp.float32)
    for _ in range(4):
        y2 = y * y
        y = 0.2 * (4.0 * y + um / (y2 * y2))
    return jnp.where(u > 0.0, y, 0.0)


def _stage(cls_t, bbox_t, gtb_t, stride):
    n_cls, n = cls_t.shape
    n_gt = gtb_t.shape[1]
    rows = 16
    cls_steps = n_cls // rows
    grid = cls_steps + 1

    def body(c_ref, b_ref, g_ref, o_ref):
        i = pl.program_id(0)

        @pl.when(i < cls_steps)
        def _():
            for k in range(rows):
                o_ref[pl.ds(k * stride, n)] = c_ref[k, :]

        @pl.when(i == cls_steps)
        def _():
            for k in range(4):
                o_ref[pl.ds(k * stride, n)] = b_ref[k, :]
                o_ref[pl.ds((8 + k) * stride, n_gt)] = g_ref[k, :]

    return pl.pallas_call(
        body,
        grid=(grid,),
        in_specs=[
            pl.BlockSpec((rows, n),
                         lambda i: (jnp.minimum(i, cls_steps - 1), 0)),
            pl.BlockSpec((4, n), lambda i: (0, 0)),
            pl.BlockSpec((4, n_gt), lambda i: (0, 0)),
        ],
        out_specs=pl.BlockSpec((rows * stride,), lambda i: (i,)),
        out_shape=jax.ShapeDtypeStruct((grid * rows * stride,), jnp.float32),
    )(cls_t, bbox_t, gtb_t)


def _make_sc_kernel(n, stride, n_pos, n_gt, n_cls):
    chunk = -(-n // (_NW * _L)) * _L
    tail = n - chunk * (_NW - 1)
    assert 0 < tail <= chunk and tail % 8 == 0
    ppw = n_pos // _NW
    mesh = plsc.VectorSubcoreMesh(
        core_axis_name="c", subcore_axis_name="s", num_cores=1)

    @functools.partial(
        pl.kernel,
        out_type=jax.ShapeDtypeStruct((n,), jnp.float32),
        mesh=mesh,
        compiler_params=pltpu.CompilerParams(
            needs_layout_passes=False, use_tc_tiling_on_sc=False),
        scratch_types=[
            pltpu.VMEM((chunk,), jnp.float32),
            pltpu.VMEM((ppw,), jnp.int32),
            pltpu.VMEM((4 * ppw,), jnp.int32),
            pltpu.VMEM((4 * ppw,), jnp.float32),
            pltpu.VMEM((ppw * n_cls,), jnp.int32),
            pltpu.VMEM((ppw * n_cls,), jnp.float32),
            pltpu.VMEM((4 * 104,), jnp.float32),
            pltpu.VMEM((n_gt,), jnp.int32),
            pltpu.VMEM((ppw,), jnp.float32),
            pltpu.SemaphoreType.DMA,
            pltpu.SemaphoreType.DMA,
            pltpu.SemaphoreType.DMA,
        ],
    )
    def sc_kernel(staged_flat, positive_inds, gt_labels,
                  out,
                  zero_v, idx_v, bidx_v, boxc_v, cidx_v, clsf_v, gtb_v,
                  gtl_v, val_v, sem0, sem1, sem2):
        w = lax.axis_index("s")

        def zbody(i, carry):
            zero_v[pl.ds(i * _L, _L)] = jnp.zeros((_L,), jnp.float32)
            return carry
        lax.fori_loop(0, chunk // _L, zbody, 0)

        @pl.when(w < _NW - 1)
        def _():
            pltpu.sync_copy(zero_v, out.at[pl.ds(w * chunk, chunk)])

        @pl.when(w == _NW - 1)
        def _():
            pltpu.sync_copy(zero_v.at[pl.ds(0, tail)],
                            out.at[pl.ds((_NW - 1) * chunk, tail)])

        plsc.subcore_barrier()

        cp_gtl = pltpu.async_copy(gt_labels, gtl_v, sem2)
        gt_copies = [
            pltpu.async_copy(staged_flat.at[pl.ds((88 + c) * stride, n_gt)],
                             gtb_v.at[pl.ds(c * 104, n_gt)], sem2)
            for c in range(4)]
        pltpu.sync_copy(positive_inds.at[pl.ds(w * ppw, ppw)], idx_v)
        idx_lo = idx_v[pl.ds(0, _L)]
        idx_hi = idx_v[pl.ds(_L, _L)]
        for c in range(4):
            base = (n_cls + c) * stride
            bidx_v[pl.ds(c * ppw, _L)] = idx_lo + base
            bidx_v[pl.ds(c * ppw + _L, _L)] = idx_hi + base
        cp_box = pltpu.async_copy(staged_flat.at[bidx_v], boxc_v, sem0)
        def cbody(c, carry):
            cidx_v[pl.ds(c * ppw, _L)] = idx_lo + c * stride
            cidx_v[pl.ds(c * ppw + _L, _L)] = idx_hi + c * stride
            return carry
        lax.fori_loop(0, n_cls, cbody, 0)
        cls_copies = []
        n_chunk = ppw * n_cls // 128
        for k in range(n_chunk):
            cls_copies.append(pltpu.async_copy(
                staged_flat.at[cidx_v.at[pl.ds(k * 128, 128)]],
                clsf_v.at[pl.ds(k * 128, 128)], sem1))
        cp_gtl.wait()
        for cp in gt_copies:
            cp.wait()
        cp_box.wait()
        for cp in cls_copies:
            cp.wait()

        lanes = lax.iota(jnp.int32, _L)
        zeros_i = jnp.zeros((_L,), jnp.int32)
        for pb in range(ppw // _L):
            ax1 = boxc_v[pl.ds(0 * ppw + pb * _L, _L)]
            ay1 = boxc_v[pl.ds(1 * ppw + pb * _L, _L)]
            ax2 = boxc_v[pl.ds(2 * ppw + pb * _L, _L)]
            ay2 = boxc_v[pl.ds(3 * ppw + pb * _L, _L)]
            area1 = (ax2 - ax1) * (ay2 - ay1)
            pidx = lanes + pb * _L

            def one_gt(g):
                lab = plsc.load_gather(gtl_v, [zeros_i + g])
                gx1 = plsc.load_gather(gtb_v, [zeros_i + g])
                gy1 = plsc.load_gather(gtb_v, [zeros_i + (104 + g)])
                gx2 = plsc.load_gather(gtb_v, [zeros_i + (2 * 104 + g)])
                gy2 = plsc.load_gather(gtb_v, [zeros_i + (3 * 104 + g)])
                sraw = plsc.load_gather(clsf_v, [lab * ppw + pidx])
                s = 1.0 / (1.0 + jnp.exp(-sraw))
                area2 = (gx2 - gx1) * (gy2 - gy1)
                iw = jnp.maximum(
                    jnp.minimum(ax2, gx2) - jnp.maximum(ax1, gx1), 0.0)
                ih = jnp.maximum(
                    jnp.minimum(ay2, gy2) - jnp.maximum(ay1, gy1), 0.0)
                inter = iw * ih
                union = jnp.maximum(area1 + area2 - inter, 1e-6)
                iou = inter / union
                iou2 = iou * iou
                return s * (iou2 * iou2)

            def gbody(i, m):
                us = [one_gt(i * 4 + j) for j in range(4)]
                u01 = jnp.maximum(us[0], us[1])
                u23 = jnp.maximum(us[2], us[3])
                return jnp.maximum(m, jnp.maximum(u01, u23))

            assert n_gt % 4 == 0
            m = lax.fori_loop(0, n_gt // 4, gbody,
                              jnp.zeros((_L,), jnp.float32))
            val_v[pl.ds(pb * _L, _L)] = _fifth_root(m)

        pltpu.async_copy(val_v, out.at[idx_v], sem0).wait()

    return sc_kernel


def kernel(bboxes, cls_scores, bbox_preds, gt_bboxes, bbox_levels,
           positive_inds, gt_labels):
    del bboxes, bbox_levels
    n, n_cls = cls_scores.shape
    n_pos = positive_inds.shape[0]
    n_gt = gt_bboxes.shape[0]
    stride = -(-n // 128) * 128
    staged = _stage(cls_scores.T, bbox_preds[:, :4].T, gt_bboxes[:, :4].T,
                    stride)
    sc_kernel = _make_sc_kernel(n, stride, n_pos, n_gt, n_cls)
    return sc_kernel(staged,
                     positive_inds.astype(jnp.int32),
                     gt_labels.astype(jnp.int32))
